# Initial kernel scaffold; baseline (speedup 1.0000x reference)
import jax
import jax.numpy as jnp
from jax.experimental import pallas as pl


def kernel(x, edge_index, edge_attr, batch, node_W, node_b, edge_W, edge_b, W1a, b1a, W1b, b1b, W2a, b2a, W2b, b2b):
    raise NotImplementedError("write your pallas kernel here")


# speedup vs baseline: 5.1883x; 5.1883x over previous
import functools

import jax
import jax.numpy as jnp
from jax import lax
from jax.experimental import pallas as pl
from jax.experimental.pallas import tpu as pltpu
from jax.experimental.pallas import tpu_sc as plsc

_NC = 2
_NS = 16
_NW = _NC * _NS
_C = 80


def _edge_agg(htab, src, dst, a, ew):
    N, H = htab.shape
    E = src.shape[0]
    epw = E // _NW
    nchunk = epw // _C
    KH = H // 16
    zrows = 128
    npad = -(-N // (_NS * zrows)) * (_NS * zrows)
    rows_per_tile = npad // _NS
    nz = rows_per_tile // zrows

    mesh = plsc.VectorSubcoreMesh(core_axis_name="c", subcore_axis_name="s")

    @functools.partial(
        pl.kernel,
        mesh=mesh,
        out_type=jax.ShapeDtypeStruct((_NC, npad, H), jnp.float32),
        scratch_types=[
            pltpu.VMEM((_C,), jnp.int32),
            pltpu.VMEM((_C,), jnp.float32),
            pltpu.VMEM((_C,), jnp.int32),
            pltpu.VMEM((_C, H), jnp.float32),
            pltpu.VMEM((zrows, H), jnp.float32),
            pltpu.VMEM((H,), jnp.float32),
            pltpu.VMEM_SHARED((npad, H), jnp.float32),
            pltpu.SemaphoreType.DMA,
        ],
    )
    def k(htab_h, src_h, dst_h, a_h, ew_h, out_h,
          srcc_v, ac_v, dstc_v, rows_v, zb_v, ew_v, agg_s, sem):
        cid = lax.axis_index("c")
        sid = lax.axis_index("s")
        wid = sid * _NC + cid
        base_e = wid * epw

        pltpu.sync_copy(ew_h, ew_v)
        ew_r = [ew_v[pl.ds(j * 16, 16)] for j in range(KH)]
        z16 = jnp.zeros((16,), jnp.float32)

        def zb_row(r, carry):
            for j in range(KH):
                zb_v[r, pl.ds(j * 16, 16)] = z16
            return carry
        lax.fori_loop(0, zrows, zb_row, 0)
        base = sid * rows_per_tile
        for i in range(nz):
            pltpu.sync_copy(zb_v, agg_s.at[pl.ds(base + i * zrows, zrows)])
        plsc.subcore_barrier()

        def chunk(t, carry):
            off = base_e + t * _C
            pltpu.sync_copy(src_h.at[pl.ds(off, _C)], srcc_v)
            pltpu.sync_copy(a_h.at[pl.ds(off, _C)], ac_v)
            pltpu.sync_copy(dst_h.at[pl.ds(off, _C)], dstc_v)
            pltpu.async_copy(htab_h.at[srcc_v], rows_v, sem).wait()

            def group(g, c2):
                av = ac_v[pl.ds(g * 16, 16)]
                for j in range(16):
                    aj = av[j]
                    row = g * 16 + j
                    for q in range(KH):
                        r = rows_v[row, pl.ds(q * 16, 16)]
                        rows_v[row, pl.ds(q * 16, 16)] = jnp.maximum(
                            r + aj * ew_r[q], 0.0)
                return c2
            lax.fori_loop(0, _C // 16, group, 0)

            pltpu.sync_copy(rows_v, agg_s.at[dstc_v], add=True)
            return carry
        lax.fori_loop(0, nchunk, chunk, 0)
        plsc.subcore_barrier()

        for i in range(nz):
            sl = pl.ds(base + i * zrows, zrows)
            pltpu.sync_copy(agg_s.at[sl], zb_v)
            pltpu.sync_copy(zb_v, out_h.at[cid, sl])

    return k(htab, src, dst, a, ew)


def _enc_tc(x, node_W, cvec):
    N = x.shape[0]
    H = node_W.shape[1]

    def body(x_ref, w_ref, c_ref, o_ref):
        o_ref[...] = x_ref[...] * w_ref[...] + c_ref[...]

    return pl.pallas_call(
        body,
        out_shape=jax.ShapeDtypeStruct((N, H), jnp.float32),
    )(x, node_W, cvec)


def _mlp_tc(htab, ebm, parts, Wa, ba, Wb, bb, add_eb):
    N, H = htab.shape

    def body(t_ref, e_ref, p_ref, wa_ref, ba_ref, wb_ref, bb_ref, o_ref):
        z = (t_ref[...] - e_ref[...] + p_ref[0, pl.ds(0, N)]
             + p_ref[1, pl.ds(0, N)])
        y = jnp.maximum(
            jnp.dot(z, wa_ref[...], preferred_element_type=jnp.float32)
            + ba_ref[...], 0.0)
        h = jnp.maximum(
            jnp.dot(y, wb_ref[...], preferred_element_type=jnp.float32)
            + bb_ref[...], 0.0)
        if add_eb:
            h = h + e_ref[...]
        o_ref[...] = h

    return pl.pallas_call(
        body,
        out_shape=jax.ShapeDtypeStruct((N, H), jnp.float32),
    )(htab, ebm, parts, Wa, ba, Wb, bb)


def _final_tc(htab, ebm, parts, Wa, ba, Wb, bb, batch2, G):
    N, H = htab.shape

    def body(t_ref, e_ref, p_ref, wa_ref, ba_ref, wb_ref, bb_ref, b_ref,
             o_ref):
        z = (t_ref[...] - e_ref[...] + p_ref[0, pl.ds(0, N)]
             + p_ref[1, pl.ds(0, N)])
        y = jnp.maximum(
            jnp.dot(z, wa_ref[...], preferred_element_type=jnp.float32)
            + ba_ref[...], 0.0)
        h = jnp.maximum(
            jnp.dot(y, wb_ref[...], preferred_element_type=jnp.float32)
            + bb_ref[...], 0.0)
        gids = lax.broadcasted_iota(jnp.int32, (G, N), 0)
        oh = (gids == b_ref[...]).astype(jnp.float32)
        sums = jnp.dot(oh, h, preferred_element_type=jnp.float32)
        counts = jnp.sum(oh, axis=1, keepdims=True)
        o_ref[...] = sums / jnp.maximum(counts, 1.0)

    return pl.pallas_call(
        body,
        out_shape=jax.ShapeDtypeStruct((G, H), jnp.float32),
    )(htab, ebm, parts, Wa, ba, Wb, bb, batch2)


@jax.jit
def kernel(x, edge_index, edge_attr, batch, node_W, node_b, edge_W, edge_b,
           W1a, b1a, W1b, b1b, W2a, b2a, W2b, bb2):
    N = x.shape[0]
    E = edge_index.shape[1]
    H = node_W.shape[1]
    G = 64

    src = edge_index[0]
    dst = edge_index[1]
    a = edge_attr.astype(jnp.float32).reshape(E)
    ew = edge_W.astype(jnp.float32).reshape(H)
    ebm = edge_b.astype(jnp.float32).reshape(1, H)
    cvec = (node_b + edge_b).astype(jnp.float32).reshape(1, H)
    batch2 = batch.reshape(1, N)

    htab1 = _enc_tc(x.astype(jnp.float32), node_W, cvec)
    parts1 = _edge_agg(htab1, src, dst, a, ew)
    htab2 = _mlp_tc(htab1, ebm, parts1, W1a, b1a.reshape(1, H),
                    W1b, b1b.reshape(1, H), True)
    parts2 = _edge_agg(htab2, src, dst, a, ew)
    out = _final_tc(htab2, ebm, parts2, W2a, b2a.reshape(1, H),
                    W2b, bb2.reshape(1, H), batch2, G)
    return out
